# Initial kernel scaffold; baseline (speedup 1.0000x reference)
import jax
import jax.numpy as jnp
from jax.experimental import pallas as pl


def kernel(indices, table):
    raise NotImplementedError("write your pallas kernel here")


# speedup vs baseline: 7.2290x; 7.2290x over previous
import functools

import jax
import jax.numpy as jnp
from jax import lax
from jax.experimental import pallas as pl
from jax.experimental.pallas import tpu as pltpu
from jax.experimental.pallas import tpu_sc as plsc

B = 16384
H = 20
D = 64
NC = 2
NS = 16
NW = NC * NS
BPW = B // NW
CB = 32
IPC = CB * H
IVL = 128
NIV = IPC // IVL
NCH = BPW // CB
LANES = 16


def _body(idx_hbm, table_hbm, out_hbm, idx_v, rows_v, out_v, sem):
    c = lax.axis_index("c")
    s = lax.axis_index("s")
    wid = s * NC + c
    pltpu.sync_copy(idx_hbm.at[wid], idx_v)

    def chunk(ci, carry):
        b0 = wid * BPW + ci * CB
        cps = []
        for j in range(NIV):
            cp = pltpu.make_async_copy(
                table_hbm.at[idx_v.at[ci * NIV + j]],
                rows_v.at[pl.ds(j * IVL, IVL)],
                sem,
            )
            cp.start()
            cps.append(cp)
        for cp in cps:
            cp.wait()

        def acc_b(b, carry2):
            for j in range(D // LANES):
                a = rows_v[b * H, pl.ds(j * LANES, LANES)]
                for l in range(1, H):
                    a = a + rows_v[b * H + l, pl.ds(j * LANES, LANES)]
                out_v[b, pl.ds(j * LANES, LANES)] = a * (1.0 / H)
            return carry2

        lax.fori_loop(0, CB, acc_b, 0, unroll=False)
        pltpu.sync_copy(out_v, out_hbm.at[pl.ds(b0, CB)])
        return carry

    lax.fori_loop(0, NCH, chunk, 0, unroll=False)


_mesh = plsc.VectorSubcoreMesh(core_axis_name="c", subcore_axis_name="s")

_sc_call = functools.partial(
    pl.kernel,
    out_type=jax.ShapeDtypeStruct((B, D), jnp.float32),
    mesh=_mesh,
    scratch_types=[
        pltpu.VMEM((NCH * NIV, IVL), jnp.int32),
        pltpu.VMEM((IPC, D), jnp.float32),
        pltpu.VMEM((CB, D), jnp.float32),
        pltpu.SemaphoreType.DMA,
    ],
    compiler_params=pltpu.CompilerParams(use_tc_tiling_on_sc=False),
)(_body)


def kernel(indices, table):
    idx = indices.astype(jnp.int32).reshape(NW, NCH * NIV, IVL)
    return _sc_call(idx, table)


# speedup vs baseline: 10.0948x; 1.3964x over previous
import functools

import jax
import jax.numpy as jnp
from jax import lax
from jax.experimental import pallas as pl
from jax.experimental.pallas import tpu as pltpu
from jax.experimental.pallas import tpu_sc as plsc

B = 16384
H = 20
D = 64
NC = 2
NS = 16
NW = NC * NS
BPW = B // NW
IVL = 128
GPW = BPW // IVL
LANES = 16


def _body(idx_hbm, table_hbm, out_hbm, idx_v, acc_v, sem):
    c = lax.axis_index("c")
    s = lax.axis_index("s")
    wid = s * NC + c
    g0 = wid * GPW
    pltpu.sync_copy(idx_hbm.at[pl.ds(g0, GPW)], idx_v)

    cps = []
    for j in range(GPW):
        cps.append(pltpu.async_copy(
            table_hbm.at[idx_v.at[j, 0]],
            acc_v.at[pl.ds(j * IVL, IVL)],
            sem,
        ))
    for cp in cps:
        cp.wait()

    def add_pass(l, carry):
        cps2 = []
        for j in range(GPW):
            cps2.append(pltpu.async_copy(
                table_hbm.at[idx_v.at[j, l]],
                acc_v.at[pl.ds(j * IVL, IVL)],
                sem,
                add=True,
            ))
        for cp in cps2:
            cp.wait()
        return carry

    lax.fori_loop(1, H, add_pass, 0, unroll=False)

    def scale_row(r, carry):
        for j in range(D // LANES):
            acc_v[r, pl.ds(j * LANES, LANES)] = (
                acc_v[r, pl.ds(j * LANES, LANES)] * (1.0 / H))
        return carry

    lax.fori_loop(0, BPW, scale_row, 0, unroll=False)
    pltpu.sync_copy(acc_v, out_hbm.at[pl.ds(wid * BPW, BPW)])


_mesh = plsc.VectorSubcoreMesh(core_axis_name="c", subcore_axis_name="s")

_sc_call = functools.partial(
    pl.kernel,
    out_type=jax.ShapeDtypeStruct((B, D), jnp.float32),
    mesh=_mesh,
    scratch_types=[
        pltpu.VMEM((GPW, H, IVL), jnp.int32),
        pltpu.VMEM((BPW, D), jnp.float32),
        pltpu.SemaphoreType.DMA,
    ],
    compiler_params=pltpu.CompilerParams(use_tc_tiling_on_sc=False),
)(_body)


def kernel(indices, table):
    idx = indices.astype(jnp.int32).reshape(B // IVL, IVL, H)
    idx = idx.transpose(0, 2, 1)
    return _sc_call(idx, table)
